# Initial kernel scaffold; baseline (speedup 1.0000x reference)
import jax
import jax.numpy as jnp
from jax.experimental import pallas as pl


def kernel(x_e, pos_e, edge_index_e, edge_attr_e, batch_node, batch_edge, W1, b1, W2, b2):
    raise NotImplementedError("write your pallas kernel here")


# speedup vs baseline: 2.9715x; 2.9715x over previous
import jax
import jax.numpy as jnp
from jax import lax
from jax.experimental import pallas as pl
from jax.experimental.pallas import tpu as pltpu
from jax.experimental.pallas import tpu_sc as plsc

NUM_NODES = 10000
HIDDEN = 256
NUM_GRAPHS = 64
NUM_WORKERS = 32
ROWS_PER_WORKER = 320
GROUPS = 4
GROUP_ROWS = 80
CNT_W = 16


def _pool_body(x_hbm, ids_hbm, sums_hbm, cnts_hbm,
               rows_v, i0, i1, i2, i3, ones_v, zrow_v, zcnt_v, acc_sh, cnt_sh):
    cid = lax.axis_index("c")
    sid = lax.axis_index("s")
    wid = sid * 2 + cid
    base = wid * ROWS_PER_WORKER
    idx_refs = (i0, i1, i2, i3)

    one16 = jnp.ones((16,), jnp.float32)
    zero16 = jnp.zeros((16,), jnp.float32)
    for i in range(GROUP_ROWS):
        ones_v[i, :] = one16
    for i in range(4):
        for j in range(HIDDEN // 16):
            zrow_v[i, pl.ds(j * 16, 16)] = zero16
        zcnt_v[i, :] = zero16

    pltpu.sync_copy(zrow_v, acc_sh.at[pl.ds(sid * 4, 4)])
    pltpu.sync_copy(zcnt_v, cnt_sh.at[pl.ds(sid * 4, 4)])

    for j in range(GROUPS):
        pltpu.sync_copy(ids_hbm.at[wid, j], idx_refs[j])
    plsc.subcore_barrier()

    for j in range(GROUPS):
        @pl.when(base + j * GROUP_ROWS < NUM_NODES)
        def _(j=j):
            pltpu.sync_copy(x_hbm.at[pl.ds(base + j * GROUP_ROWS, GROUP_ROWS)],
                            rows_v.at[pl.ds(j * GROUP_ROWS, GROUP_ROWS)])
            pltpu.sync_copy(rows_v.at[pl.ds(j * GROUP_ROWS, GROUP_ROWS)],
                            acc_sh.at[idx_refs[j]], add=True)
            pltpu.sync_copy(ones_v, cnt_sh.at[idx_refs[j]], add=True)

    plsc.subcore_barrier()

    @pl.when(sid == 0)
    def _():
        pltpu.sync_copy(acc_sh, sums_hbm.at[cid])
        pltpu.sync_copy(cnt_sh, cnts_hbm.at[cid])


@jax.jit
def _sc_pool(x_e, ids_p):
    mesh = plsc.VectorSubcoreMesh(core_axis_name="c", subcore_axis_name="s")
    f = pl.kernel(
        _pool_body,
        out_type=[
            jax.ShapeDtypeStruct((2, NUM_GRAPHS, HIDDEN), jnp.float32),
            jax.ShapeDtypeStruct((2, NUM_GRAPHS, CNT_W), jnp.float32),
        ],
        mesh=mesh,
        scratch_types=[
            pltpu.VMEM((ROWS_PER_WORKER, HIDDEN), jnp.float32),
            pltpu.VMEM((GROUP_ROWS,), jnp.int32),
            pltpu.VMEM((GROUP_ROWS,), jnp.int32),
            pltpu.VMEM((GROUP_ROWS,), jnp.int32),
            pltpu.VMEM((GROUP_ROWS,), jnp.int32),
            pltpu.VMEM((GROUP_ROWS, CNT_W), jnp.float32),
            pltpu.VMEM((4, HIDDEN), jnp.float32),
            pltpu.VMEM((4, CNT_W), jnp.float32),
            pltpu.VMEM_SHARED((NUM_GRAPHS, HIDDEN), jnp.float32),
            pltpu.VMEM_SHARED((NUM_GRAPHS, CNT_W), jnp.float32),
        ],
        compiler_params=pltpu.CompilerParams(use_tc_tiling_on_sc=False),
    )
    return f(x_e, ids_p)


def _head_body(s_ref, c_ref, w1_ref, b1_ref, w2_ref, b2_ref, o_ref):
    s = s_ref[0] + s_ref[1]
    c = c_ref[0] + c_ref[1]
    cnt = jnp.maximum(c[:, 0:1], 1.0)
    mean = s / cnt
    h = jnp.dot(mean, w1_ref[...], preferred_element_type=jnp.float32)
    h = jnp.maximum(h + b1_ref[...], 0.0)
    o_ref[...] = jnp.dot(h, w2_ref[...], preferred_element_type=jnp.float32) + b2_ref[...]


@jax.jit
def _tc_head(sums, cnts, W1, b1, W2, b2):
    return pl.pallas_call(
        _head_body,
        out_shape=jax.ShapeDtypeStruct((NUM_GRAPHS, 10), jnp.float32),
    )(sums, cnts, W1, b1, W2, b2)


def kernel(x_e, pos_e, edge_index_e, edge_attr_e, batch_node, batch_edge,
           W1, b1, W2, b2):
    ids = batch_node.astype(jnp.int32)
    pad = NUM_WORKERS * ROWS_PER_WORKER - NUM_NODES
    ids_p = jnp.concatenate([ids, jnp.zeros((pad,), jnp.int32)])
    ids_p = ids_p.reshape(NUM_WORKERS, GROUPS, GROUP_ROWS)
    sums, cnts = _sc_pool(x_e, ids_p)
    return _tc_head(sums, cnts, W1, b1.reshape(1, -1), W2, b2.reshape(1, -1))


# speedup vs baseline: 3.2915x; 1.1077x over previous
import jax
import jax.numpy as jnp
from jax import lax
from jax.experimental import pallas as pl
from jax.experimental.pallas import tpu as pltpu
from jax.experimental.pallas import tpu_sc as plsc

NUM_NODES = 10000
HIDDEN = 256
NUM_GRAPHS = 64
NUM_WORKERS = 32
ROWS_PER_WORKER = 320
GROUPS = 4
GROUP_ROWS = 80
CNT_W = 16
ACC_ROWS = 80
TRASH = 64
PAD_IDS = NUM_WORKERS * ROWS_PER_WORKER - NUM_NODES
LAST_BASE = NUM_NODES - GROUP_ROWS


def _pool_body(x_hbm, ids_hbm, sums_hbm, cnts_hbm,
               rows_v, i0, i1, i2, i3, ones_v, zrow_v, zcnt_v,
               acc_sh, cnt_sh, sem_i, sem_r, sem_s):
    cid = lax.axis_index("c")
    sid = lax.axis_index("s")
    wid = sid * 2 + cid
    base = wid * ROWS_PER_WORKER
    idx_refs = (i0, i1, i2, i3)

    id_copies = [
        pltpu.async_copy(ids_hbm.at[pl.ds(base + j * GROUP_ROWS, GROUP_ROWS)],
                         idx_refs[j], sem_i)
        for j in range(GROUPS)
    ]
    bases = [jnp.minimum(base + j * GROUP_ROWS, LAST_BASE) for j in range(GROUPS)]
    row_copies = [None] * GROUPS
    row_copies[0] = pltpu.async_copy(
        x_hbm.at[pl.ds(bases[0], GROUP_ROWS)],
        rows_v.at[pl.ds(0, GROUP_ROWS)], sem_r)

    one16 = jnp.ones((16,), jnp.float32)
    zero16 = jnp.zeros((16,), jnp.float32)
    for i in range(GROUP_ROWS):
        ones_v[i, :] = one16
    for i in range(5):
        for j in range(HIDDEN // 16):
            zrow_v[i, pl.ds(j * 16, 16)] = zero16
        zcnt_v[i, :] = zero16

    pltpu.sync_copy(zrow_v, acc_sh.at[pl.ds(sid * 5, 5)])
    pltpu.sync_copy(zcnt_v, cnt_sh.at[pl.ds(sid * 5, 5)])
    plsc.subcore_barrier()

    for c in id_copies:
        c.wait()

    scatters = []
    for j in range(GROUPS):
        row_copies[j].wait()
        if j + 1 < GROUPS:
            row_copies[j + 1] = pltpu.async_copy(
                x_hbm.at[pl.ds(bases[j + 1], GROUP_ROWS)],
                rows_v.at[pl.ds((j + 1) * GROUP_ROWS, GROUP_ROWS)], sem_r)
        scatters.append(pltpu.async_copy(
            rows_v.at[pl.ds(j * GROUP_ROWS, GROUP_ROWS)],
            acc_sh.at[idx_refs[j]], sem_s, add=True))
        scatters.append(pltpu.async_copy(
            ones_v, cnt_sh.at[idx_refs[j]], sem_s, add=True))
    for s in scatters:
        s.wait()

    plsc.subcore_barrier()

    @pl.when(sid == 0)
    def _():
        pltpu.sync_copy(acc_sh.at[pl.ds(0, NUM_GRAPHS)], sums_hbm.at[cid])
        pltpu.sync_copy(cnt_sh.at[pl.ds(0, NUM_GRAPHS)], cnts_hbm.at[cid])


@jax.jit
def _sc_pool(x_e, ids_p):
    mesh = plsc.VectorSubcoreMesh(core_axis_name="c", subcore_axis_name="s")
    f = pl.kernel(
        _pool_body,
        out_type=[
            jax.ShapeDtypeStruct((2, NUM_GRAPHS, HIDDEN), jnp.float32),
            jax.ShapeDtypeStruct((2, NUM_GRAPHS, CNT_W), jnp.float32),
        ],
        mesh=mesh,
        scratch_types=[
            pltpu.VMEM((ROWS_PER_WORKER, HIDDEN), jnp.float32),
            pltpu.VMEM((GROUP_ROWS,), jnp.int32),
            pltpu.VMEM((GROUP_ROWS,), jnp.int32),
            pltpu.VMEM((GROUP_ROWS,), jnp.int32),
            pltpu.VMEM((GROUP_ROWS,), jnp.int32),
            pltpu.VMEM((GROUP_ROWS, CNT_W), jnp.float32),
            pltpu.VMEM((5, HIDDEN), jnp.float32),
            pltpu.VMEM((5, CNT_W), jnp.float32),
            pltpu.VMEM_SHARED((ACC_ROWS, HIDDEN), jnp.float32),
            pltpu.VMEM_SHARED((ACC_ROWS, CNT_W), jnp.float32),
            pltpu.SemaphoreType.DMA,
            pltpu.SemaphoreType.DMA,
            pltpu.SemaphoreType.DMA,
        ],
        compiler_params=pltpu.CompilerParams(use_tc_tiling_on_sc=False),
    )
    return f(x_e, ids_p)


def _head_body(s_ref, c_ref, w1_ref, b1_ref, w2_ref, b2_ref, o_ref):
    s = s_ref[0] + s_ref[1]
    c = c_ref[0] + c_ref[1]
    cnt = jnp.maximum(c[:, 0:1], 1.0)
    mean = s / cnt
    h = jnp.dot(mean, w1_ref[...], preferred_element_type=jnp.float32)
    h = jnp.maximum(h + b1_ref[...], 0.0)
    o_ref[...] = jnp.dot(h, w2_ref[...], preferred_element_type=jnp.float32) + b2_ref[...]


@jax.jit
def _tc_head(sums, cnts, W1, b1, W2, b2):
    return pl.pallas_call(
        _head_body,
        out_shape=jax.ShapeDtypeStruct((NUM_GRAPHS, 10), jnp.float32),
    )(sums, cnts, W1, b1, W2, b2)


def kernel(x_e, pos_e, edge_index_e, edge_attr_e, batch_node, batch_edge,
           W1, b1, W2, b2):
    ids = batch_node.astype(jnp.int32)
    ids_p = jnp.concatenate([ids, jnp.full((PAD_IDS,), TRASH, jnp.int32)])
    sums, cnts = _sc_pool(x_e, ids_p)
    return _tc_head(sums, cnts, W1, b1.reshape(1, -1), W2, b2.reshape(1, -1))


# speedup vs baseline: 4.2389x; 1.2878x over previous
import jax
import jax.numpy as jnp
from jax import lax
from jax.experimental import pallas as pl
from jax.experimental.pallas import tpu as pltpu
from jax.experimental.pallas import tpu_sc as plsc

NUM_NODES = 10000
HIDDEN = 256
NUM_GRAPHS = 64
NUM_WORKERS = 32
NUM_PIECES = NUM_NODES * 2
PIECES_PER_WORKER = NUM_PIECES // NUM_WORKERS
GROUPS = 5
GROUP_PIECES = PIECES_PER_WORKER // GROUPS
CNT_W = 16
ACC_ROWS = 2 * NUM_GRAPHS


def _pool_body(x_hbm, ids_hbm, sums_hbm, cnts_hbm,
               rows_v, i0, i1, i2, i3, i4, ones_v, zrow_v, zcnt_v,
               acc_sh, cnt_sh, sem_i, sem_r, sem_s):
    cid = lax.axis_index("c")
    sid = lax.axis_index("s")
    wid = sid * 2 + cid
    idx_refs = (i0, i1, i2, i3, i4)

    id_copies = [
        pltpu.async_copy(ids_hbm.at[wid * GROUPS + j], idx_refs[j], sem_i)
        for j in range(GROUPS)
    ]
    row_copies = [None] * GROUPS
    row_copies[0] = pltpu.async_copy(
        x_hbm.at[pl.ds(wid * PIECES_PER_WORKER, GROUP_PIECES)],
        rows_v.at[pl.ds(0, GROUP_PIECES)], sem_r)

    one16 = jnp.ones((16,), jnp.float32)
    zero16 = jnp.zeros((16,), jnp.float32)
    for i in range(GROUP_PIECES):
        ones_v[i, :] = one16
    for i in range(8):
        for j in range(128 // 16):
            zrow_v[i, pl.ds(j * 16, 16)] = zero16
        zcnt_v[i, :] = zero16

    pltpu.sync_copy(zrow_v, acc_sh.at[pl.ds(sid * 8, 8)])
    pltpu.sync_copy(zcnt_v, cnt_sh.at[pl.ds(sid * 8, 8)])
    plsc.subcore_barrier()

    for c in id_copies:
        c.wait()

    scatters = []
    for j in range(GROUPS):
        row_copies[j].wait()
        if j + 1 < GROUPS:
            row_copies[j + 1] = pltpu.async_copy(
                x_hbm.at[pl.ds(wid * PIECES_PER_WORKER + (j + 1) * GROUP_PIECES,
                               GROUP_PIECES)],
                rows_v.at[pl.ds((j + 1) * GROUP_PIECES, GROUP_PIECES)], sem_r)
        scatters.append(pltpu.async_copy(
            rows_v.at[pl.ds(j * GROUP_PIECES, GROUP_PIECES)],
            acc_sh.at[idx_refs[j]], sem_s, add=True))
        scatters.append(pltpu.async_copy(
            ones_v, cnt_sh.at[idx_refs[j]], sem_s, add=True))
    for s in scatters:
        s.wait()

    plsc.subcore_barrier()

    @pl.when(sid == 0)
    def _():
        pltpu.sync_copy(acc_sh, sums_hbm.at[cid])
        pltpu.sync_copy(cnt_sh, cnts_hbm.at[cid])


@jax.jit
def _sc_pool(x4, ids2):
    mesh = plsc.VectorSubcoreMesh(core_axis_name="c", subcore_axis_name="s")
    f = pl.kernel(
        _pool_body,
        out_type=[
            jax.ShapeDtypeStruct((2, ACC_ROWS, 128), jnp.float32),
            jax.ShapeDtypeStruct((2, ACC_ROWS, CNT_W), jnp.float32),
        ],
        mesh=mesh,
        scratch_types=[
            pltpu.VMEM((PIECES_PER_WORKER, 128), jnp.float32),
            pltpu.VMEM((GROUP_PIECES,), jnp.int32),
            pltpu.VMEM((GROUP_PIECES,), jnp.int32),
            pltpu.VMEM((GROUP_PIECES,), jnp.int32),
            pltpu.VMEM((GROUP_PIECES,), jnp.int32),
            pltpu.VMEM((GROUP_PIECES,), jnp.int32),
            pltpu.VMEM((GROUP_PIECES, CNT_W), jnp.float32),
            pltpu.VMEM((8, 128), jnp.float32),
            pltpu.VMEM((8, CNT_W), jnp.float32),
            pltpu.VMEM_SHARED((ACC_ROWS, 128), jnp.float32),
            pltpu.VMEM_SHARED((ACC_ROWS, CNT_W), jnp.float32),
            pltpu.SemaphoreType.DMA,
            pltpu.SemaphoreType.DMA,
            pltpu.SemaphoreType.DMA,
        ],
        compiler_params=pltpu.CompilerParams(use_tc_tiling_on_sc=False),
    )
    return f(x4, ids2)


def _head_body(s_ref, c_ref, w1_ref, b1_ref, w2_ref, b2_ref, o_ref):
    s = s_ref[0] + s_ref[1]
    c = c_ref[0] + c_ref[1]
    cnt = jnp.maximum(c[:, 0:1], 1.0)
    s = s / cnt
    r_iota = lax.broadcasted_iota(jnp.int32, (NUM_GRAPHS, ACC_ROWS), 0)
    c_iota = lax.broadcasted_iota(jnp.int32, (NUM_GRAPHS, ACC_ROWS), 1)
    e0 = (c_iota == 2 * r_iota).astype(jnp.float32)
    e1 = (c_iota == 2 * r_iota + 1).astype(jnp.float32)
    me = jnp.dot(e0, s, preferred_element_type=jnp.float32)
    mo = jnp.dot(e1, s, preferred_element_type=jnp.float32)
    h = (jnp.dot(me, w1_ref[0:128, :], preferred_element_type=jnp.float32)
         + jnp.dot(mo, w1_ref[128:256, :], preferred_element_type=jnp.float32)
         + b1_ref[...])
    h = jnp.maximum(h, 0.0)
    o_ref[...] = jnp.dot(h, w2_ref[...], preferred_element_type=jnp.float32) + b2_ref[...]


@jax.jit
def _tc_head(sums, cnts, W1, b1, W2, b2):
    return pl.pallas_call(
        _head_body,
        out_shape=jax.ShapeDtypeStruct((NUM_GRAPHS, 10), jnp.float32),
    )(sums, cnts, W1, b1, W2, b2)


def kernel(x_e, pos_e, edge_index_e, edge_attr_e, batch_node, batch_edge,
           W1, b1, W2, b2):
    x4 = x_e.reshape(1250, 8, 2, 128).transpose(0, 2, 1, 3).reshape(NUM_PIECES, 128)
    ids = batch_node.astype(jnp.int32)
    ids_r = ids.reshape(1250, 1, 8)
    ids2 = 2 * ids_r + jnp.arange(2, dtype=jnp.int32).reshape(1, 2, 1)
    ids2 = ids2.reshape(NUM_WORKERS * GROUPS, GROUP_PIECES)
    sums, cnts = _sc_pool(x4, ids2)
    return _tc_head(sums, cnts, W1, b1.reshape(1, -1), W2, b2.reshape(1, -1))


# speedup vs baseline: 4.3757x; 1.0323x over previous
import jax
import jax.numpy as jnp
from jax import lax
from jax.experimental import pallas as pl
from jax.experimental.pallas import tpu as pltpu
from jax.experimental.pallas import tpu_sc as plsc

NUM_NODES = 10000
HIDDEN = 256
NUM_GRAPHS = 64
NUM_WORKERS = 32
NUM_PIECES = NUM_NODES * 2
PPW = NUM_PIECES // NUM_WORKERS
GROUPS = 5
GP = 128
IDS_LEN = 328
ACC_ROWS = 144
TRASH = 128
CNT_LEN = 144


def _pool_body(x_hbm, ids_hbm, zeros_hbm, sums_hbm, cnts_hbm,
               rows_v, ids_v, i0, i1, i2, i3, i4, cnt_v, cnt_all, cout_v,
               acc_sh, cnt_stage, sem_i, sem_r, sem_s, sem_z):
    cid = lax.axis_index("c")
    sid = lax.axis_index("s")
    wid = sid * 2 + cid
    idx_refs = (i0, i1, i2, i3, i4)
    p0 = wid * PPW

    ids_base = jnp.minimum(8 * (p0 // 16), NUM_NODES - IDS_LEN)
    ids_cp = pltpu.async_copy(ids_hbm.at[pl.ds(ids_base, IDS_LEN)], ids_v, sem_i)

    qbases = [jnp.minimum(p0 + j * GP, NUM_PIECES - GP) for j in range(GROUPS)]
    row_copies = [None] * GROUPS
    row_copies[0] = pltpu.async_copy(
        x_hbm.at[pl.ds(qbases[0], GP)], rows_v.at[pl.ds(0, GP)], sem_r)

    zcp = pltpu.async_copy(zeros_hbm, acc_sh.at[pl.ds(sid * 9, 9)], sem_z)

    zero16 = jnp.zeros((16,), jnp.float32)
    one16 = jnp.ones((16,), jnp.float32)
    for k in range(CNT_LEN // 16):
        cnt_v[0, pl.ds(k * 16, 16)] = zero16

    ids_cp.wait()
    lanes = lax.iota(jnp.int32, 16)
    zero16i = jnp.zeros((16,), jnp.int32)
    for j in range(GROUPS):
        lo = p0 + j * GP
        hi = p0 + PPW
        for k in range(GP // 16):
            qv = qbases[j] + (k * 16) + lanes
            lrow = ((qv >> 4) << 3) + (qv & 7) - ids_base
            idv = plsc.load_gather(ids_v, [lrow])
            idx = 2 * idv + ((qv >> 3) & 1)
            valid = (qv >= lo) & (qv < hi)
            idx = jnp.where(valid, idx, TRASH)
            idx_refs[j][pl.ds(k * 16, 16)] = idx
            plsc.addupdate_scatter(cnt_v, [zero16i, idx],
                                   jnp.where(valid, one16, zero16))

    zcp.wait()
    plsc.subcore_barrier()

    scatters = []
    for j in range(GROUPS):
        row_copies[j].wait()
        if j + 1 < GROUPS:
            row_copies[j + 1] = pltpu.async_copy(
                x_hbm.at[pl.ds(qbases[j + 1], GP)],
                rows_v.at[pl.ds((j + 1) * GP, GP)], sem_r)
        scatters.append(pltpu.async_copy(
            rows_v.at[pl.ds(j * GP, GP)],
            acc_sh.at[idx_refs[j]], sem_s, add=True))
    for s in scatters:
        s.wait()

    pltpu.sync_copy(cnt_v, cnt_stage.at[pl.ds(sid, 1)])
    plsc.subcore_barrier()

    @pl.when(sid == 0)
    def _():
        pltpu.sync_copy(cnt_stage, cnt_all)
        lanes_ = lax.iota(jnp.int32, 16)
        zc = jnp.zeros((16,), jnp.int32)
        for k in range(8):
            tot = cnt_all[0, pl.ds(k * 16, 16)]
            for t in range(1, 16):
                tot = tot + cnt_all[t, pl.ds(k * 16, 16)]
            plsc.store_scatter(cout_v, [k * 16 + lanes_, zc], tot)
        pltpu.sync_copy(acc_sh.at[pl.ds(0, 2 * NUM_GRAPHS)], sums_hbm.at[cid])
        pltpu.sync_copy(cout_v, cnts_hbm.at[cid])


@jax.jit
def _sc_pool(x4, ids, zeros9):
    mesh = plsc.VectorSubcoreMesh(core_axis_name="c", subcore_axis_name="s")
    f = pl.kernel(
        _pool_body,
        out_type=[
            jax.ShapeDtypeStruct((2, 2 * NUM_GRAPHS, 128), jnp.float32),
            jax.ShapeDtypeStruct((2, 128, 128), jnp.float32),
        ],
        mesh=mesh,
        scratch_types=[
            pltpu.VMEM((GROUPS * GP, 128), jnp.float32),
            pltpu.VMEM((IDS_LEN,), jnp.int32),
            pltpu.VMEM((GP,), jnp.int32),
            pltpu.VMEM((GP,), jnp.int32),
            pltpu.VMEM((GP,), jnp.int32),
            pltpu.VMEM((GP,), jnp.int32),
            pltpu.VMEM((GP,), jnp.int32),
            pltpu.VMEM((1, CNT_LEN), jnp.float32),
            pltpu.VMEM((16, CNT_LEN), jnp.float32),
            pltpu.VMEM((128, 128), jnp.float32),
            pltpu.VMEM_SHARED((ACC_ROWS, 128), jnp.float32),
            pltpu.VMEM_SHARED((16, CNT_LEN), jnp.float32),
            pltpu.SemaphoreType.DMA,
            pltpu.SemaphoreType.DMA,
            pltpu.SemaphoreType.DMA,
            pltpu.SemaphoreType.DMA,
        ],
        compiler_params=pltpu.CompilerParams(
            use_tc_tiling_on_sc=False, needs_layout_passes=False),
    )
    return f(x4, ids, zeros9)


def _head_body(s_ref, c_ref, w1_ref, b1_ref, w2_ref, b2_ref, o_ref):
    s = s_ref[0] + s_ref[1]
    c = c_ref[0, :, 0:1] + c_ref[1, :, 0:1]
    s = s / jnp.maximum(c, 1.0)
    r_iota = lax.broadcasted_iota(jnp.int32, (NUM_GRAPHS, 2 * NUM_GRAPHS), 0)
    c_iota = lax.broadcasted_iota(jnp.int32, (NUM_GRAPHS, 2 * NUM_GRAPHS), 1)
    e0 = (c_iota == 2 * r_iota).astype(jnp.float32)
    e1 = (c_iota == 2 * r_iota + 1).astype(jnp.float32)
    me = jnp.dot(e0, s, preferred_element_type=jnp.float32)
    mo = jnp.dot(e1, s, preferred_element_type=jnp.float32)
    h = (jnp.dot(me, w1_ref[0:128, :], preferred_element_type=jnp.float32)
         + jnp.dot(mo, w1_ref[128:256, :], preferred_element_type=jnp.float32)
         + b1_ref[...])
    h = jnp.maximum(h, 0.0)
    o_ref[...] = jnp.dot(h, w2_ref[...], preferred_element_type=jnp.float32) + b2_ref[...]


@jax.jit
def _tc_head(sums, cnts, W1, b1, W2, b2):
    return pl.pallas_call(
        _head_body,
        out_shape=jax.ShapeDtypeStruct((NUM_GRAPHS, 10), jnp.float32),
    )(sums, cnts, W1, b1, W2, b2)


def kernel(x_e, pos_e, edge_index_e, edge_attr_e, batch_node, batch_edge,
           W1, b1, W2, b2):
    x4 = x_e.reshape(1250, 8, 2, 128).transpose(0, 2, 1, 3).reshape(NUM_PIECES, 128)
    ids = batch_node.astype(jnp.int32)
    zeros9 = jnp.zeros((9, 128), jnp.float32)
    sums, cnts = _sc_pool(x4, ids, zeros9)
    return _tc_head(sums, cnts, W1, b1.reshape(1, -1), W2, b2.reshape(1, -1))


# speedup vs baseline: 4.4694x; 1.0214x over previous
import jax
import jax.numpy as jnp
from jax import lax
from jax.experimental import pallas as pl
from jax.experimental.pallas import tpu as pltpu
from jax.experimental.pallas import tpu_sc as plsc

NUM_NODES = 10000
HIDDEN = 256
NUM_GRAPHS = 64
NUM_WORKERS = 32
NUM_PIECES = NUM_NODES * 2
PPW = NUM_PIECES // NUM_WORKERS
GROUPS = 5
GP = 128
IDS_LEN = 328
ACC_ROWS = 144
TRASH = 128
CNT_LEN = 144


def _pool_body(x_hbm, ids_hbm, zeros_hbm, sums_hbm, cnts_hbm,
               rows_v, ids_v, i0, i1, i2, i3, i4, cnt_v, cnt_all, cout_v,
               acc_sh, cnt_stage, sem_i, sem_r, sem_s, sem_z):
    cid = lax.axis_index("c")
    sid = lax.axis_index("s")
    wid = sid * 2 + cid
    idx_refs = (i0, i1, i2, i3, i4)
    p0 = wid * PPW

    ids_base = jnp.minimum(8 * (p0 // 16), NUM_NODES - IDS_LEN)
    ids_cp = pltpu.async_copy(ids_hbm.at[pl.ds(ids_base, IDS_LEN)], ids_v, sem_i)

    qbases = [jnp.minimum(p0 + j * GP, NUM_PIECES - GP) for j in range(GROUPS)]
    row_copies = [None] * GROUPS
    row_copies[0] = pltpu.async_copy(
        x_hbm.at[pl.ds(qbases[0], GP)], rows_v.at[pl.ds(0, GP)], sem_r)

    zcp = pltpu.async_copy(zeros_hbm, acc_sh.at[pl.ds(sid * 9, 9)], sem_z)

    zero16 = jnp.zeros((16,), jnp.float32)
    one16 = jnp.ones((16,), jnp.float32)
    for k in range(CNT_LEN // 16):
        cnt_v[0, pl.ds(k * 16, 16)] = zero16

    ids_cp.wait()
    lanes = lax.iota(jnp.int32, 16)
    zero16i = jnp.zeros((16,), jnp.int32)

    def build_idx(j):
        lo = p0 + j * GP
        hi = p0 + PPW
        for k in range(GP // 16):
            qv = qbases[j] + (k * 16) + lanes
            lrow = ((qv >> 4) << 3) + (qv & 7) - ids_base
            idv = plsc.load_gather(ids_v, [lrow])
            idx = 2 * idv + ((qv >> 3) & 1)
            valid = (qv >= lo) & (qv < hi)
            idx = jnp.where(valid, idx, TRASH)
            idx_refs[j][pl.ds(k * 16, 16)] = idx
            plsc.addupdate_scatter(cnt_v, [zero16i, idx],
                                   jnp.where(valid, one16, zero16))

    build_idx(0)
    zcp.wait()
    plsc.subcore_barrier()

    scatters = []
    for j in range(GROUPS):
        row_copies[j].wait()
        if j + 1 < GROUPS:
            row_copies[j + 1] = pltpu.async_copy(
                x_hbm.at[pl.ds(qbases[j + 1], GP)],
                rows_v.at[pl.ds((j + 1) * GP, GP)], sem_r)
        scatters.append(pltpu.async_copy(
            rows_v.at[pl.ds(j * GP, GP)],
            acc_sh.at[idx_refs[j]], sem_s, add=True))
        if j + 1 < GROUPS:
            build_idx(j + 1)

    pltpu.sync_copy(cnt_v, cnt_stage.at[pl.ds(sid, 1)])
    for s in scatters:
        s.wait()
    plsc.subcore_barrier()

    @pl.when(sid == 0)
    def _():
        pltpu.sync_copy(acc_sh.at[pl.ds(0, 2 * NUM_GRAPHS)], sums_hbm.at[cid])

    @pl.when(sid == 1)
    def _():
        pltpu.sync_copy(cnt_stage, cnt_all)
        lanes_ = lax.iota(jnp.int32, 16)
        zc = jnp.zeros((16,), jnp.int32)
        for k in range(8):
            tot = cnt_all[0, pl.ds(k * 16, 16)]
            for t in range(1, 16):
                tot = tot + cnt_all[t, pl.ds(k * 16, 16)]
            plsc.store_scatter(cout_v, [k * 16 + lanes_, zc], tot)
        pltpu.sync_copy(cout_v, cnts_hbm.at[cid])


@jax.jit
def _sc_pool(x4, ids, zeros9):
    mesh = plsc.VectorSubcoreMesh(core_axis_name="c", subcore_axis_name="s")
    f = pl.kernel(
        _pool_body,
        out_type=[
            jax.ShapeDtypeStruct((2, 2 * NUM_GRAPHS, 128), jnp.float32),
            jax.ShapeDtypeStruct((2, 128, 128), jnp.float32),
        ],
        mesh=mesh,
        scratch_types=[
            pltpu.VMEM((GROUPS * GP, 128), jnp.float32),
            pltpu.VMEM((IDS_LEN,), jnp.int32),
            pltpu.VMEM((GP,), jnp.int32),
            pltpu.VMEM((GP,), jnp.int32),
            pltpu.VMEM((GP,), jnp.int32),
            pltpu.VMEM((GP,), jnp.int32),
            pltpu.VMEM((GP,), jnp.int32),
            pltpu.VMEM((1, CNT_LEN), jnp.float32),
            pltpu.VMEM((16, CNT_LEN), jnp.float32),
            pltpu.VMEM((128, 128), jnp.float32),
            pltpu.VMEM_SHARED((ACC_ROWS, 128), jnp.float32),
            pltpu.VMEM_SHARED((16, CNT_LEN), jnp.float32),
            pltpu.SemaphoreType.DMA,
            pltpu.SemaphoreType.DMA,
            pltpu.SemaphoreType.DMA,
            pltpu.SemaphoreType.DMA,
        ],
        compiler_params=pltpu.CompilerParams(
            use_tc_tiling_on_sc=False, needs_layout_passes=False),
    )
    return f(x4, ids, zeros9)


def _head_body(s_ref, c_ref, w1_ref, b1_ref, w2_ref, b2_ref, o_ref):
    s = s_ref[0] + s_ref[1]
    c = c_ref[0, :, 0:1] + c_ref[1, :, 0:1]
    s = s / jnp.maximum(c, 1.0)
    r_iota = lax.broadcasted_iota(jnp.int32, (NUM_GRAPHS, 2 * NUM_GRAPHS), 0)
    c_iota = lax.broadcasted_iota(jnp.int32, (NUM_GRAPHS, 2 * NUM_GRAPHS), 1)
    e0 = (c_iota == 2 * r_iota).astype(jnp.float32)
    e1 = (c_iota == 2 * r_iota + 1).astype(jnp.float32)
    me = jnp.dot(e0, s, preferred_element_type=jnp.float32)
    mo = jnp.dot(e1, s, preferred_element_type=jnp.float32)
    h = (jnp.dot(me, w1_ref[0:128, :], preferred_element_type=jnp.float32)
         + jnp.dot(mo, w1_ref[128:256, :], preferred_element_type=jnp.float32)
         + b1_ref[...])
    h = jnp.maximum(h, 0.0)
    o_ref[...] = jnp.dot(h, w2_ref[...], preferred_element_type=jnp.float32) + b2_ref[...]


@jax.jit
def _tc_head(sums, cnts, W1, b1, W2, b2):
    return pl.pallas_call(
        _head_body,
        out_shape=jax.ShapeDtypeStruct((NUM_GRAPHS, 10), jnp.float32),
    )(sums, cnts, W1, b1, W2, b2)


def kernel(x_e, pos_e, edge_index_e, edge_attr_e, batch_node, batch_edge,
           W1, b1, W2, b2):
    x4 = x_e.reshape(1250, 8, 2, 128).transpose(0, 2, 1, 3).reshape(NUM_PIECES, 128)
    ids = batch_node.astype(jnp.int32)
    zeros9 = jnp.zeros((9, 128), jnp.float32)
    sums, cnts = _sc_pool(x4, ids, zeros9)
    return _tc_head(sums, cnts, W1, b1.reshape(1, -1), W2, b2.reshape(1, -1))


# speedup vs baseline: 4.6777x; 1.0466x over previous
import jax
import jax.numpy as jnp
from jax import lax
from jax.experimental import pallas as pl
from jax.experimental.pallas import tpu as pltpu
from jax.experimental.pallas import tpu_sc as plsc

NUM_NODES = 10000
HIDDEN = 256
NUM_GRAPHS = 64
NUM_WORKERS = 32
NUM_PIECES = NUM_NODES * 2
PPW = NUM_PIECES // NUM_WORKERS
GROUPS = 5
GP = 128
IDS_LEN = 328
ACC_ROWS = 144
TRASH = 128
CNT_LEN = 144


def _pool_body(x_hbm, ids_hbm, sums_hbm, cnts_hbm,
               rows_v, ids_v, i0, i1, i2, i3, i4, cnt_v, cnt_all, cout_v,
               zrow_v, acc_sh, cnt_stage, sem_i, sem_r, sem_s):
    cid = lax.axis_index("c")
    sid = lax.axis_index("s")
    wid = sid * 2 + cid
    idx_refs = (i0, i1, i2, i3, i4)
    p0 = wid * PPW

    ids_base = jnp.minimum(8 * (p0 // 16), NUM_NODES - IDS_LEN)
    ids_cp = pltpu.async_copy(ids_hbm.at[pl.ds(ids_base, IDS_LEN)], ids_v, sem_i)

    qbases = [jnp.minimum(p0 + j * GP, NUM_PIECES - GP) for j in range(GROUPS)]
    row_copies = [
        pltpu.async_copy(x_hbm.at[pl.ds(qbases[j], GP)],
                         rows_v.at[pl.ds(j * GP, GP)], sem_r)
        for j in range(GROUPS)
    ]

    zero16 = jnp.zeros((16,), jnp.float32)
    one16 = jnp.ones((16,), jnp.float32)
    for k in range(CNT_LEN // 16):
        cnt_v[0, pl.ds(k * 16, 16)] = zero16
    for i in range(9):
        for k in range(8):
            zrow_v[i, pl.ds(k * 16, 16)] = zero16
    zcp = pltpu.async_copy(zrow_v, acc_sh.at[pl.ds(sid * 9, 9)], sem_s)

    ids_cp.wait()
    lanes = lax.iota(jnp.int32, 16)
    zero16i = jnp.zeros((16,), jnp.int32)

    def build_idx(j):
        lo = p0 + j * GP
        hi = p0 + PPW
        for k in range(GP // 16):
            qv = qbases[j] + (k * 16) + lanes
            lrow = ((qv >> 4) << 3) + (qv & 7) - ids_base
            idv = plsc.load_gather(ids_v, [lrow])
            idx = 2 * idv + ((qv >> 3) & 1)
            valid = (qv >= lo) & (qv < hi)
            idx = jnp.where(valid, idx, TRASH)
            idx_refs[j][pl.ds(k * 16, 16)] = idx
            plsc.addupdate_scatter(cnt_v, [zero16i, idx],
                                   jnp.where(valid, one16, zero16))

    build_idx(0)
    zcp.wait()
    plsc.subcore_barrier()

    scatters = []
    for j in range(GROUPS):
        row_copies[j].wait()
        scatters.append(pltpu.async_copy(
            rows_v.at[pl.ds(j * GP, GP)],
            acc_sh.at[idx_refs[j]], sem_s, add=True))
        if j + 1 < GROUPS:
            build_idx(j + 1)

    pltpu.sync_copy(cnt_v, cnt_stage.at[pl.ds(sid, 1)])
    for s in scatters:
        s.wait()
    plsc.subcore_barrier()

    @pl.when(sid == 0)
    def _():
        pltpu.sync_copy(acc_sh.at[pl.ds(0, 2 * NUM_GRAPHS)], sums_hbm.at[cid])

    @pl.when(sid == 1)
    def _():
        pltpu.sync_copy(cnt_stage, cnt_all)
        lanes_ = lax.iota(jnp.int32, 16)
        zc = jnp.zeros((16,), jnp.int32)
        for k in range(8):
            tot = cnt_all[0, pl.ds(k * 16, 16)]
            for t in range(1, 16):
                tot = tot + cnt_all[t, pl.ds(k * 16, 16)]
            plsc.store_scatter(cout_v, [k * 16 + lanes_, zc], tot)
        pltpu.sync_copy(cout_v, cnts_hbm.at[cid])


@jax.jit
def _sc_pool(x4, ids):
    mesh = plsc.VectorSubcoreMesh(core_axis_name="c", subcore_axis_name="s")
    f = pl.kernel(
        _pool_body,
        out_type=[
            jax.ShapeDtypeStruct((2, 2 * NUM_GRAPHS, 128), jnp.float32),
            jax.ShapeDtypeStruct((2, 128, 128), jnp.float32),
        ],
        mesh=mesh,
        scratch_types=[
            pltpu.VMEM((GROUPS * GP, 128), jnp.float32),
            pltpu.VMEM((IDS_LEN,), jnp.int32),
            pltpu.VMEM((GP,), jnp.int32),
            pltpu.VMEM((GP,), jnp.int32),
            pltpu.VMEM((GP,), jnp.int32),
            pltpu.VMEM((GP,), jnp.int32),
            pltpu.VMEM((GP,), jnp.int32),
            pltpu.VMEM((1, CNT_LEN), jnp.float32),
            pltpu.VMEM((16, CNT_LEN), jnp.float32),
            pltpu.VMEM((128, 128), jnp.float32),
            pltpu.VMEM((9, 128), jnp.float32),
            pltpu.VMEM_SHARED((ACC_ROWS, 128), jnp.float32),
            pltpu.VMEM_SHARED((16, CNT_LEN), jnp.float32),
            pltpu.SemaphoreType.DMA,
            pltpu.SemaphoreType.DMA,
            pltpu.SemaphoreType.DMA,
        ],
        compiler_params=pltpu.CompilerParams(
            use_tc_tiling_on_sc=False, needs_layout_passes=False),
    )
    return f(x4, ids)


def _head_body(s_ref, c_ref, w1_ref, b1_ref, w2_ref, b2_ref, o_ref):
    s = s_ref[0] + s_ref[1]
    c = c_ref[0, :, 0:1] + c_ref[1, :, 0:1]
    s = s / jnp.maximum(c, 1.0)
    r_iota = lax.broadcasted_iota(jnp.int32, (NUM_GRAPHS, 2 * NUM_GRAPHS), 0)
    c_iota = lax.broadcasted_iota(jnp.int32, (NUM_GRAPHS, 2 * NUM_GRAPHS), 1)
    e0 = (c_iota == 2 * r_iota).astype(jnp.float32)
    e1 = (c_iota == 2 * r_iota + 1).astype(jnp.float32)
    me = jnp.dot(e0, s, preferred_element_type=jnp.float32)
    mo = jnp.dot(e1, s, preferred_element_type=jnp.float32)
    h = (jnp.dot(me, w1_ref[0:128, :], preferred_element_type=jnp.float32)
         + jnp.dot(mo, w1_ref[128:256, :], preferred_element_type=jnp.float32)
         + b1_ref[...])
    h = jnp.maximum(h, 0.0)
    o_ref[...] = jnp.dot(h, w2_ref[...], preferred_element_type=jnp.float32) + b2_ref[...]


@jax.jit
def _tc_head(sums, cnts, W1, b1, W2, b2):
    return pl.pallas_call(
        _head_body,
        out_shape=jax.ShapeDtypeStruct((NUM_GRAPHS, 10), jnp.float32),
    )(sums, cnts, W1, b1, W2, b2)


def kernel(x_e, pos_e, edge_index_e, edge_attr_e, batch_node, batch_edge,
           W1, b1, W2, b2):
    x4 = x_e.reshape(1250, 8, 2, 128).transpose(0, 2, 1, 3).reshape(NUM_PIECES, 128)
    ids = batch_node.astype(jnp.int32)
    sums, cnts = _sc_pool(x4, ids)
    return _tc_head(sums, cnts, W1, b1.reshape(1, -1), W2, b2.reshape(1, -1))
